# baseline (speedup 1.0000x reference)
import functools

import jax
import jax.numpy as jnp
from jax import lax
from jax.experimental import pallas as pl
from jax.experimental.pallas import tpu as pltpu
from jax.experimental.pallas import tpu_sc as plsc

N_NODES = 10000
D_IN = 128
N_EDGES = 320000

NC = 2
NS = 16
NW = NC * NS

CHUNK = 128
EPT = 10240
NCHUNK = EPT // CHUNK
E_PAD = EPT * NW
NP = 10240
RPT = NP // NS
NBUF = 4
RBUF = 8
DW = 130

_mesh = plsc.VectorSubcoreMesh(core_axis_name="c", subcore_axis_name="s")


@functools.partial(
    pl.kernel,
    out_type=jax.ShapeDtypeStruct((NC, NP, DW), jnp.bfloat16),
    mesh=_mesh,
    compiler_params=pltpu.CompilerParams(use_tc_tiling_on_sc=False),
    scratch_types=[
        pltpu.VMEM((NCHUNK, CHUNK), jnp.int32),
        pltpu.VMEM((RBUF, CHUNK), jnp.int32),
        pltpu.VMEM((NBUF, CHUNK, DW), jnp.bfloat16),
        pltpu.VMEM_SHARED((NP, DW), jnp.bfloat16),
        [pltpu.SemaphoreType.DMA] * NBUF,
        [pltpu.SemaphoreType.DMA] * RBUF,
    ],
)
def _sc_aggregate(x_hbm, rows_hbm, cols_hbm, zeros2_hbm, outf_hbm,
                  cols_v, rbuf, gbuf, accf_sh, gsems, rsems):
    c = lax.axis_index("c")
    s = lax.axis_index("s")
    w = s * NC + c
    lo = s * RPT

    pltpu.sync_copy(zeros2_hbm, accf_sh.at[pl.ds(lo, RPT)])
    pltpu.sync_copy(cols_hbm.at[w], cols_v)
    plsc.subcore_barrier()

    def start_rows(j, rslot):
        pltpu.async_copy(rows_hbm.at[w, j], rbuf.at[rslot], rsems[rslot])

    def start_gather(j, rslot, gslot):
        pltpu.make_async_copy(rows_hbm.at[w, j], rbuf.at[rslot],
                              rsems[rslot]).wait()
        pltpu.async_copy(x_hbm.at[rbuf.at[rslot]], gbuf.at[gslot],
                         gsems[gslot])

    def scatter(j, rslot, gslot):
        pltpu.make_async_copy(x_hbm.at[rbuf.at[rslot]], gbuf.at[gslot],
                              gsems[gslot]).wait()
        pltpu.sync_copy(gbuf.at[gslot], accf_sh.at[cols_v.at[j]], add=True)

    for j in range(RBUF):
        start_rows(j, j)
    for j in range(NBUF):
        start_gather(j, j % RBUF, j % NBUF)

    def ring(i, carry):
        for b8 in range(RBUF):
            j = i * RBUF + b8
            scatter(j, b8, b8 % NBUF)
            start_gather(j + NBUF, (b8 + NBUF) % RBUF, b8 % NBUF)
            start_rows(j + RBUF, b8)
        return carry

    lax.fori_loop(0, NCHUNK // RBUF - 1, ring, 0)
    for j in range(NCHUNK - RBUF, NCHUNK):
        scatter(j, j % RBUF, j % NBUF)
        if j + NBUF < NCHUNK:
            start_gather(j + NBUF, (j + NBUF) % RBUF, (j + NBUF) % NBUF)

    plsc.subcore_barrier()
    pltpu.sync_copy(accf_sh.at[pl.ds(lo, RPT)], outf_hbm.at[c, pl.ds(lo, RPT)])


def _tc_dense_body(aggf_ref, x_ref, w_ref, b_ref, o_ref):
    a = (aggf_ref[0].astype(jnp.float32) +
         aggf_ref[1].astype(jnp.float32))
    feat = a[:, :D_IN]
    deg = jnp.maximum(a[:, D_IN:D_IN + 1], 1.0)
    h = feat / deg + x_ref[...]
    o = jnp.dot(h, w_ref[...], preferred_element_type=jnp.float32,
                precision=lax.Precision.HIGHEST)
    o_ref[...] = jnp.maximum(o + b_ref[...], 0.0)


_BLK = 1024


def kernel(x, edge_index, W, b):
    row = edge_index[0]
    col = edge_index[1]
    ept_real = N_EDGES // NW
    pad_t = EPT - ept_real
    pad_cols = jnp.broadcast_to(N_NODES + jnp.arange(pad_t, dtype=jnp.int32),
                                (NW, pad_t))
    rows = jnp.concatenate(
        [row.reshape(NW, ept_real), jnp.zeros((NW, pad_t), jnp.int32)],
        axis=1).reshape(NW, NCHUNK, CHUNK)
    cols = jnp.concatenate(
        [col.reshape(NW, ept_real), pad_cols],
        axis=1).reshape(NW, NCHUNK, CHUNK)
    xb = x.astype(jnp.bfloat16)
    xab = jnp.concatenate(
        [xb, jnp.ones((N_NODES, 1), jnp.bfloat16),
         jnp.zeros((N_NODES, DW - D_IN - 1), jnp.bfloat16)], axis=1)
    zeros2 = jnp.zeros((RPT, DW), jnp.bfloat16)

    aggf = _sc_aggregate(xab, rows, cols, zeros2)

    grid = (N_NODES + _BLK - 1) // _BLK
    out = pl.pallas_call(
        _tc_dense_body,
        grid=(grid,),
        in_specs=[
            pl.BlockSpec((NC, _BLK, DW), lambda i: (0, i, 0)),
            pl.BlockSpec((_BLK, D_IN), lambda i: (i, 0)),
            pl.BlockSpec((D_IN, D_IN), lambda i: (0, 0)),
            pl.BlockSpec((1, D_IN), lambda i: (0, 0)),
        ],
        out_specs=pl.BlockSpec((_BLK, D_IN), lambda i: (i, 0)),
        out_shape=jax.ShapeDtypeStruct((N_NODES, D_IN), jnp.float32),
    )(aggf, x, W, b.reshape(1, D_IN))
    return out


# speedup vs baseline: 1.2493x; 1.2493x over previous
import functools

import jax
import jax.numpy as jnp
from jax import lax
from jax.experimental import pallas as pl
from jax.experimental.pallas import tpu as pltpu
from jax.experimental.pallas import tpu_sc as plsc

N_NODES = 10000
D_IN = 128
N_EDGES = 320000

NC = 2
NS = 16
NW = NC * NS

CHUNK = 128
EPT = 10240
NCHUNK = EPT // CHUNK
E_PAD = EPT * NW
NP = 10240
RPT = NP // NS
NBUF = 4
RBUF = 8

_mesh = plsc.VectorSubcoreMesh(core_axis_name="c", subcore_axis_name="s")


@functools.partial(
    pl.kernel,
    out_type=(
        jax.ShapeDtypeStruct((NC, NP, D_IN), jnp.bfloat16),
        jax.ShapeDtypeStruct((NC, NP), jnp.float32),
    ),
    mesh=_mesh,
    compiler_params=pltpu.CompilerParams(use_tc_tiling_on_sc=False),
    scratch_types=[
        pltpu.VMEM((NCHUNK, CHUNK), jnp.int32),
        pltpu.VMEM((RBUF, CHUNK), jnp.int32),
        pltpu.VMEM((NBUF, CHUNK, D_IN), jnp.bfloat16),
        pltpu.VMEM((CHUNK,), jnp.float32),
        pltpu.VMEM_SHARED((NP, D_IN), jnp.bfloat16),
        pltpu.VMEM_SHARED((NP,), jnp.float32),
        [pltpu.SemaphoreType.DMA] * NBUF,
        [pltpu.SemaphoreType.DMA] * RBUF,
        [pltpu.SemaphoreType.DMA] * NBUF,
    ],
)
def _sc_aggregate(x_hbm, rows_hbm, cols_hbm, zeros2_hbm, ones_hbm, zeros1_hbm,
                  outf_hbm, outd_hbm,
                  cols_v, rbuf, gbuf, ones_v, accf_sh, accd_sh,
                  gsems, rsems, dsems):
    c = lax.axis_index("c")
    s = lax.axis_index("s")
    w = s * NC + c
    lo = s * RPT

    pltpu.sync_copy(zeros2_hbm, accf_sh.at[pl.ds(lo, RPT)])
    pltpu.sync_copy(zeros1_hbm, accd_sh.at[pl.ds(lo, RPT)])
    pltpu.sync_copy(cols_hbm.at[w], cols_v)
    pltpu.sync_copy(ones_hbm, ones_v)
    plsc.subcore_barrier()

    def start_rows(j, rslot):
        pltpu.async_copy(rows_hbm.at[w, j], rbuf.at[rslot], rsems[rslot])

    def start_gather(j, rslot, gslot):
        pltpu.make_async_copy(rows_hbm.at[w, j], rbuf.at[rslot],
                              rsems[rslot]).wait()
        pltpu.async_copy(x_hbm.at[rbuf.at[rslot]], gbuf.at[gslot],
                         gsems[gslot])

    def scatter(j, rslot, gslot, drain_deg=True):
        pltpu.make_async_copy(x_hbm.at[rbuf.at[rslot]], gbuf.at[gslot],
                              gsems[gslot]).wait()
        if drain_deg:
            pltpu.make_async_copy(ones_v, accd_sh.at[cols_v.at[j - NBUF]],
                                  dsems[gslot]).wait()
        pltpu.async_copy(ones_v, accd_sh.at[cols_v.at[j]], dsems[gslot])
        pltpu.sync_copy(gbuf.at[gslot], accf_sh.at[cols_v.at[j]], add=True)

    for j in range(RBUF):
        start_rows(j, j)
    for j in range(NBUF):
        start_gather(j, j % RBUF, j % NBUF)

    def ring(i, carry):
        for b8 in range(RBUF):
            j = i * RBUF + b8
            scatter(j, b8, b8 % NBUF)
            start_gather(j + NBUF, (b8 + NBUF) % RBUF, b8 % NBUF)
            start_rows(j + RBUF, b8)
        return carry

    for j in range(RBUF):
        scatter(j, j, j % NBUF, drain_deg=j >= NBUF)
        start_gather(j + NBUF, (j + NBUF) % RBUF, j % NBUF)
        start_rows(j + RBUF, j)
    lax.fori_loop(1, NCHUNK // RBUF - 1, ring, 0)
    for j in range(NCHUNK - RBUF, NCHUNK):
        scatter(j, j % RBUF, j % NBUF)
        if j + NBUF < NCHUNK:
            start_gather(j + NBUF, (j + NBUF) % RBUF, (j + NBUF) % NBUF)
    for g in range(NBUF):
        pltpu.make_async_copy(ones_v, accd_sh.at[cols_v.at[NCHUNK - NBUF + g]],
                              dsems[g]).wait()

    plsc.subcore_barrier()
    pltpu.sync_copy(accf_sh.at[pl.ds(lo, RPT)], outf_hbm.at[c, pl.ds(lo, RPT)])
    pltpu.sync_copy(accd_sh.at[pl.ds(lo, RPT)], outd_hbm.at[c, pl.ds(lo, RPT)])


def _tc_dense_body(aggf_ref, deg_ref, x_ref, w_ref, b_ref, o_ref):
    i = pl.program_id(0)
    feat = (aggf_ref[0].astype(jnp.float32) +
            aggf_ref[1].astype(jnp.float32))
    d0 = deg_ref[0, pl.ds(i * _BLK, _BLK)]
    d1 = deg_ref[1, pl.ds(i * _BLK, _BLK)]
    deg = jnp.maximum(d0 + d1, 1.0)
    h = feat / deg[:, None] + x_ref[...]
    o = jnp.dot(h, w_ref[...], preferred_element_type=jnp.float32,
                precision=lax.Precision.HIGHEST)
    o_ref[...] = jnp.maximum(o + b_ref[...], 0.0)


_BLK = 1024


def kernel(x, edge_index, W, b):
    row = edge_index[0]
    col = edge_index[1]
    ept_real = N_EDGES // NW
    pad_t = EPT - ept_real
    pad_cols = jnp.broadcast_to(N_NODES + jnp.arange(pad_t, dtype=jnp.int32),
                                (NW, pad_t))
    rows = jnp.concatenate(
        [row.reshape(NW, ept_real), jnp.zeros((NW, pad_t), jnp.int32)],
        axis=1).reshape(NW, NCHUNK, CHUNK)
    cols = jnp.concatenate(
        [col.reshape(NW, ept_real), pad_cols],
        axis=1).reshape(NW, NCHUNK, CHUNK)
    xb = x.astype(jnp.bfloat16)
    zeros2 = jnp.zeros((RPT, D_IN), jnp.bfloat16)
    zeros1 = jnp.zeros((RPT,), jnp.float32)
    ones = jnp.ones((CHUNK,), jnp.float32)

    aggf, deg = _sc_aggregate(xb, rows, cols, zeros2, ones, zeros1)

    grid = (N_NODES + _BLK - 1) // _BLK
    out = pl.pallas_call(
        _tc_dense_body,
        grid=(grid,),
        in_specs=[
            pl.BlockSpec((NC, _BLK, D_IN), lambda i: (0, i, 0)),
            pl.BlockSpec((NC, NP), lambda i: (0, 0)),
            pl.BlockSpec((_BLK, D_IN), lambda i: (i, 0)),
            pl.BlockSpec((D_IN, D_IN), lambda i: (0, 0)),
            pl.BlockSpec((1, D_IN), lambda i: (0, 0)),
        ],
        out_specs=pl.BlockSpec((_BLK, D_IN), lambda i: (i, 0)),
        out_shape=jax.ShapeDtypeStruct((N_NODES, D_IN), jnp.float32),
    )(aggf, deg, x, W, b.reshape(1, D_IN))
    return out


# speedup vs baseline: 1.2525x; 1.0026x over previous
import functools

import jax
import jax.numpy as jnp
from jax import lax
from jax.experimental import pallas as pl
from jax.experimental.pallas import tpu as pltpu
from jax.experimental.pallas import tpu_sc as plsc

N_NODES = 10000
D_IN = 128
N_EDGES = 320000

NC = 2
NS = 16
NW = NC * NS

CHUNK = 128
EPT = 10240
NCHUNK = EPT // CHUNK
E_PAD = EPT * NW
NP = 10240
RPT = NP // NS
NBUF = 4
RBUF = 8

_mesh = plsc.VectorSubcoreMesh(core_axis_name="c", subcore_axis_name="s")


@functools.partial(
    pl.kernel,
    out_type=(
        jax.ShapeDtypeStruct((NC, NP, D_IN), jnp.bfloat16),
        jax.ShapeDtypeStruct((NC, NP), jnp.float32),
    ),
    mesh=_mesh,
    compiler_params=pltpu.CompilerParams(use_tc_tiling_on_sc=False),
    scratch_types=[
        pltpu.VMEM((NCHUNK, CHUNK), jnp.int32),
        pltpu.VMEM((RBUF, CHUNK), jnp.int32),
        pltpu.VMEM((NBUF, CHUNK, D_IN), jnp.bfloat16),
        pltpu.VMEM((CHUNK,), jnp.float32),
        pltpu.VMEM_SHARED((NP, D_IN), jnp.bfloat16),
        pltpu.VMEM_SHARED((NP,), jnp.float32),
        [pltpu.SemaphoreType.DMA] * NBUF,
        [pltpu.SemaphoreType.DMA] * RBUF,
    ],
)
def _sc_aggregate(x_hbm, rows_hbm, cols_hbm, zeros2_hbm, ones_hbm, zeros1_hbm,
                  outf_hbm, outd_hbm,
                  cols_v, rbuf, gbuf, ones_v, accf_sh, accd_sh, gsems, rsems):
    c = lax.axis_index("c")
    s = lax.axis_index("s")
    w = s * NC + c
    lo = s * RPT

    def start_rows(j, rslot):
        pltpu.async_copy(rows_hbm.at[w, j], rbuf.at[rslot], rsems[rslot])

    def start_gather(j, rslot, gslot):
        pltpu.make_async_copy(rows_hbm.at[w, j], rbuf.at[rslot],
                              rsems[rslot]).wait()
        pltpu.async_copy(x_hbm.at[rbuf.at[rslot]], gbuf.at[gslot],
                         gsems[gslot])

    def scatter(j, rslot, gslot):
        pltpu.make_async_copy(x_hbm.at[rbuf.at[rslot]], gbuf.at[gslot],
                              gsems[gslot]).wait()
        pltpu.sync_copy(gbuf.at[gslot], accf_sh.at[cols_v.at[j]], add=True)
        pltpu.sync_copy(ones_v, accd_sh.at[cols_v.at[j]], add=True)

    for j in range(RBUF):
        start_rows(j, j)
    for j in range(NBUF):
        start_gather(j, j % RBUF, j % NBUF)

    pltpu.sync_copy(zeros2_hbm, accf_sh.at[pl.ds(lo, RPT)])
    pltpu.sync_copy(zeros1_hbm, accd_sh.at[pl.ds(lo, RPT)])
    pltpu.sync_copy(cols_hbm.at[w], cols_v)
    pltpu.sync_copy(ones_hbm, ones_v)
    plsc.subcore_barrier()

    def ring(i, carry):
        for b8 in range(RBUF):
            j = i * RBUF + b8
            scatter(j, b8, b8 % NBUF)
            start_gather(j + NBUF, (b8 + NBUF) % RBUF, b8 % NBUF)
            start_rows(j + RBUF, b8)
        return carry

    lax.fori_loop(0, NCHUNK // RBUF - 1, ring, 0)
    for j in range(NCHUNK - RBUF, NCHUNK):
        scatter(j, j % RBUF, j % NBUF)
        if j + NBUF < NCHUNK:
            start_gather(j + NBUF, (j + NBUF) % RBUF, (j + NBUF) % NBUF)

    plsc.subcore_barrier()
    pltpu.sync_copy(accf_sh.at[pl.ds(lo, RPT)], outf_hbm.at[c, pl.ds(lo, RPT)])
    pltpu.sync_copy(accd_sh.at[pl.ds(lo, RPT)], outd_hbm.at[c, pl.ds(lo, RPT)])


def _tc_dense_body(aggf_ref, deg_ref, x_ref, w_ref, b_ref, o_ref):
    i = pl.program_id(0)
    feat = (aggf_ref[0].astype(jnp.float32) +
            aggf_ref[1].astype(jnp.float32))
    d0 = deg_ref[0, pl.ds(i * _BLK, _BLK)]
    d1 = deg_ref[1, pl.ds(i * _BLK, _BLK)]
    deg = jnp.maximum(d0 + d1, 1.0)
    h = feat / deg[:, None] + x_ref[...]
    o = jnp.dot(h, w_ref[...], preferred_element_type=jnp.float32,
                precision=lax.Precision.HIGHEST)
    o_ref[...] = jnp.maximum(o + b_ref[...], 0.0)


_BLK = 1024


def kernel(x, edge_index, W, b):
    row = edge_index[0]
    col = edge_index[1]
    ept_real = N_EDGES // NW
    pad_t = EPT - ept_real
    pad_cols = jnp.broadcast_to(N_NODES + jnp.arange(pad_t, dtype=jnp.int32),
                                (NW, pad_t))
    rows = jnp.concatenate(
        [row.reshape(NW, ept_real), jnp.zeros((NW, pad_t), jnp.int32)],
        axis=1).reshape(NW, NCHUNK, CHUNK)
    cols = jnp.concatenate(
        [col.reshape(NW, ept_real), pad_cols],
        axis=1).reshape(NW, NCHUNK, CHUNK)
    xb = x.astype(jnp.bfloat16)
    zeros2 = jnp.zeros((RPT, D_IN), jnp.bfloat16)
    zeros1 = jnp.zeros((RPT,), jnp.float32)
    ones = jnp.ones((CHUNK,), jnp.float32)

    aggf, deg = _sc_aggregate(xb, rows, cols, zeros2, ones, zeros1)

    grid = (N_NODES + _BLK - 1) // _BLK
    out = pl.pallas_call(
        _tc_dense_body,
        grid=(grid,),
        in_specs=[
            pl.BlockSpec((NC, _BLK, D_IN), lambda i: (0, i, 0)),
            pl.BlockSpec((NC, NP), lambda i: (0, 0)),
            pl.BlockSpec((_BLK, D_IN), lambda i: (i, 0)),
            pl.BlockSpec((D_IN, D_IN), lambda i: (0, 0)),
            pl.BlockSpec((1, D_IN), lambda i: (0, 0)),
        ],
        out_specs=pl.BlockSpec((_BLK, D_IN), lambda i: (i, 0)),
        out_shape=jax.ShapeDtypeStruct((N_NODES, D_IN), jnp.float32),
    )(aggf, deg, x, W, b.reshape(1, D_IN))
    return out
